# Initial kernel scaffold; baseline (speedup 1.0000x reference)
import jax
import jax.numpy as jnp
from jax.experimental import pallas as pl


def kernel(ind_2, inter):
    raise NotImplementedError("write your pallas kernel here")


# speedup vs baseline: 3.4548x; 3.4548x over previous
import functools

import jax
import jax.numpy as jnp
from jax import lax
from jax.experimental import pallas as pl
from jax.experimental.pallas import tpu as pltpu
from jax.experimental.pallas import tpu_sc as plsc

E = 320000
N = 10000
D = 128

NC = 2
NS = 16
DH = D // NC
EPT = E // NS
CH = 128
NFULL = EPT // CH
REM = EPT - NFULL * CH
RPT = N // NS

_mesh = plsc.VectorSubcoreMesh(core_axis_name="c", subcore_axis_name="s")


@functools.partial(
    pl.kernel,
    out_type=jax.ShapeDtypeStruct((N, D), jnp.float32),
    mesh=_mesh,
    compiler_params=pltpu.CompilerParams(use_tc_tiling_on_sc=False),
    scratch_types=[
        pltpu.VMEM_SHARED((N, DH), jnp.float32),
        pltpu.VMEM((CH,), jnp.int32),
        pltpu.VMEM((CH, DH), jnp.float32),
        pltpu.VMEM((REM,), jnp.int32),
        pltpu.VMEM((REM, DH), jnp.float32),
        pltpu.VMEM((RPT, DH), jnp.float32),
    ],
)
def _seg_sum(seg_hbm, inter_hbm, out_hbm, acc, idx_v, rows_v, idxr_v,
             rowsr_v, zbuf):
    c = lax.axis_index("c")
    s = lax.axis_index("s")
    col0 = c * DH

    zero16 = jnp.zeros((16,), jnp.float32)

    def zrow(i, carry):
        for j in range(DH // 16):
            zbuf[i, pl.ds(j * 16, 16)] = zero16
        return carry

    lax.fori_loop(0, RPT, zrow, 0)
    pltpu.sync_copy(zbuf, acc.at[pl.ds(s * RPT, RPT)])
    plsc.subcore_barrier()

    base = s * EPT

    def chunk(g, carry):
        off = base + g * CH
        pltpu.sync_copy(seg_hbm.at[pl.ds(off, CH)], idx_v)
        pltpu.sync_copy(inter_hbm.at[pl.ds(off, CH), pl.ds(col0, DH)], rows_v)
        pltpu.sync_copy(rows_v, acc.at[idx_v], add=True)
        return carry

    lax.fori_loop(0, NFULL, chunk, 0)

    offr = base + NFULL * CH
    pltpu.sync_copy(seg_hbm.at[pl.ds(offr, REM)], idxr_v)
    pltpu.sync_copy(inter_hbm.at[pl.ds(offr, REM), pl.ds(col0, DH)], rowsr_v)
    pltpu.sync_copy(rowsr_v, acc.at[idxr_v], add=True)

    plsc.subcore_barrier()

    r0 = s * RPT
    pltpu.sync_copy(acc.at[pl.ds(r0, RPT)], zbuf)
    pltpu.sync_copy(zbuf, out_hbm.at[pl.ds(r0, RPT), pl.ds(col0, DH)])


def kernel(ind_2, inter):
    seg = ind_2[:, 0]
    return _seg_sum(seg, inter)


# speedup vs baseline: 8.3719x; 2.4232x over previous
import functools

import jax
import jax.numpy as jnp
from jax import lax
from jax.experimental import pallas as pl
from jax.experimental.pallas import tpu as pltpu
from jax.experimental.pallas import tpu_sc as plsc

E = 320000
N = 10000
D = 128

NC = 2
NS = 16
DH = D // NC
CH = 128
SUB = 4
BC = CH * SUB
NB = 39
EPT = NB * BC
NROWS = E // CH
XTRA = NROWS - NS * NB * SUB
RPT = N // NS
ZC = 125

_mesh = plsc.VectorSubcoreMesh(core_axis_name="c", subcore_axis_name="s")


@functools.partial(
    pl.kernel,
    out_type=jax.ShapeDtypeStruct((N, D), jnp.float32),
    mesh=_mesh,
    compiler_params=pltpu.CompilerParams(use_tc_tiling_on_sc=False),
    scratch_types=[
        pltpu.VMEM_SHARED((N, DH), jnp.float32),
        pltpu.VMEM((SUB, CH), jnp.int32),
        pltpu.VMEM((BC, DH), jnp.float32),
        pltpu.VMEM((SUB, CH), jnp.int32),
        pltpu.VMEM((BC, DH), jnp.float32),
        pltpu.VMEM((ZC, DH), jnp.float32),
        pltpu.SemaphoreType.DMA,
        pltpu.SemaphoreType.DMA,
        pltpu.SemaphoreType.DMA,
        pltpu.SemaphoreType.DMA,
        pltpu.SemaphoreType.DMA,
    ],
)
def _seg_sum(seg_hbm, inter_hbm, out_hbm, acc, idx_a, rows_a, idx_b, rows_b,
             zbuf, semi_a, semr_a, semi_b, semr_b, sem_sc):
    c = lax.axis_index("c")
    s = lax.axis_index("s")
    col0 = c * DH
    row0 = s * (NB * SUB)
    eoff0 = s * EPT

    zero16 = jnp.zeros((16,), jnp.float32)

    def zrow(i, carry):
        for j in range(DH // 16):
            zbuf[i, pl.ds(j * 16, 16)] = zero16
        return carry

    lax.fori_loop(0, ZC, zrow, 0)

    def zchunk(q, carry):
        pltpu.sync_copy(zbuf, acc.at[pl.ds(s * RPT + q * ZC, ZC)])
        return carry

    lax.fori_loop(0, RPT // ZC, zchunk, 0)
    plsc.subcore_barrier()

    def loads(g, idxb, rowsb, semi, semr):
        row = row0 + g * SUB
        eoff = eoff0 + g * BC
        ci = pltpu.make_async_copy(seg_hbm.at[pl.ds(row, SUB)], idxb, semi)
        cr = pltpu.make_async_copy(
            inter_hbm.at[pl.ds(eoff, BC), pl.ds(col0, DH)], rowsb, semr)
        return ci, cr

    def fire(g, idxb, rowsb, semi, semr):
        ci, cr = loads(g, idxb, rowsb, semi, semr)
        ci.start()
        cr.start()

    def wait(g, idxb, rowsb, semi, semr):
        ci, cr = loads(g, idxb, rowsb, semi, semr)
        ci.wait()
        cr.wait()

    def scatter(idxb, rowsb):
        hs = [pltpu.async_copy(rowsb.at[pl.ds(k * CH, CH)],
                               acc.at[idxb.at[k]], sem_sc, add=True)
              for k in range(SUB)]
        for h in hs:
            h.wait()

    A = (idx_a, rows_a, semi_a, semr_a)
    B = (idx_b, rows_b, semi_b, semr_b)

    fire(0, *A)

    def body(i, carry):
        g = 2 * i
        fire(g + 1, *B)
        wait(g, *A)
        scatter(idx_a, rows_a)
        fire(g + 2, *A)
        wait(g + 1, *B)
        scatter(idx_b, rows_b)
        return carry

    lax.fori_loop(0, (NB - 1) // 2, body, 0)

    wait(NB - 1, *A)
    scatter(idx_a, rows_a)

    @pl.when(s < XTRA)
    def _tail():
        row = NS * NB * SUB + s
        eoff = row * CH
        pltpu.sync_copy(seg_hbm.at[pl.ds(row, 1)], idx_a.at[pl.ds(0, 1)])
        pltpu.sync_copy(inter_hbm.at[pl.ds(eoff, CH), pl.ds(col0, DH)],
                        rows_a.at[pl.ds(0, CH)])
        pltpu.sync_copy(rows_a.at[pl.ds(0, CH)], acc.at[idx_a.at[0]], add=True)

    plsc.subcore_barrier()

    def wchunk(q, carry):
        r0 = s * RPT + q * ZC
        pltpu.sync_copy(acc.at[pl.ds(r0, ZC)], zbuf)
        pltpu.sync_copy(zbuf, out_hbm.at[pl.ds(r0, ZC), pl.ds(col0, DH)])
        return carry

    lax.fori_loop(0, RPT // ZC, wchunk, 0)


def kernel(ind_2, inter):
    seg = ind_2[:, 0].reshape(NROWS, CH)
    return _seg_sum(seg, inter)


# speedup vs baseline: 8.5750x; 1.0243x over previous
import functools

import jax
import jax.numpy as jnp
from jax import lax
from jax.experimental import pallas as pl
from jax.experimental.pallas import tpu as pltpu
from jax.experimental.pallas import tpu_sc as plsc

E = 320000
N = 10000
D = 128

NC = 2
NS = 16
DH = D // NC
CH = 128
SUB = 4
BC = CH * SUB
NB = 39
EPT = NB * BC
NROWS = E // CH
XTRA = NROWS - NS * NB * SUB
RPT = N // NS
ZC = 125
RB2 = RPT - BC

_mesh = plsc.VectorSubcoreMesh(core_axis_name="c", subcore_axis_name="s")


@functools.partial(
    pl.kernel,
    out_type=jax.ShapeDtypeStruct((N, D), jnp.float32),
    mesh=_mesh,
    compiler_params=pltpu.CompilerParams(use_tc_tiling_on_sc=False),
    scratch_types=[
        pltpu.VMEM_SHARED((N, DH), jnp.float32),
        pltpu.VMEM((SUB, CH), jnp.int32),
        pltpu.VMEM((BC, DH), jnp.float32),
        pltpu.VMEM((SUB, CH), jnp.int32),
        pltpu.VMEM((BC, DH), jnp.float32),
        pltpu.VMEM((ZC, DH), jnp.float32),
        pltpu.SemaphoreType.DMA,
        pltpu.SemaphoreType.DMA,
        pltpu.SemaphoreType.DMA,
        pltpu.SemaphoreType.DMA,
        pltpu.SemaphoreType.DMA,
    ],
)
def _seg_sum(seg_hbm, inter_hbm, out_hbm, acc, idx_a, rows_a, idx_b, rows_b,
             zbuf, semi_a, semr_a, semi_b, semr_b, sem_sc):
    c = lax.axis_index("c")
    s = lax.axis_index("s")
    col0 = c * DH
    row0 = s * (NB * SUB)
    eoff0 = s * EPT

    def loads(g, idxb, rowsb, semi, semr):
        row = row0 + g * SUB
        eoff = eoff0 + g * BC
        ci = pltpu.make_async_copy(seg_hbm.at[pl.ds(row, SUB)], idxb, semi)
        cr = pltpu.make_async_copy(
            inter_hbm.at[pl.ds(eoff, BC), pl.ds(col0, DH)], rowsb, semr)
        return ci, cr

    def fire(g, idxb, rowsb, semi, semr):
        ci, cr = loads(g, idxb, rowsb, semi, semr)
        ci.start()
        cr.start()

    def wait(g, idxb, rowsb, semi, semr):
        ci, cr = loads(g, idxb, rowsb, semi, semr)
        ci.wait()
        cr.wait()

    def scatter(idxb, rowsb):
        hs = [pltpu.async_copy(rowsb.at[pl.ds(k * CH, CH)],
                               acc.at[idxb.at[k]], sem_sc, add=True)
              for k in range(SUB)]
        for h in hs:
            h.wait()

    A = (idx_a, rows_a, semi_a, semr_a)
    B = (idx_b, rows_b, semi_b, semr_b)

    fire(0, *A)
    fire(1, *B)

    zero16 = jnp.zeros((16,), jnp.float32)

    def zrow(i, carry):
        for j in range(DH // 16):
            zbuf[i, pl.ds(j * 16, 16)] = zero16
        return carry

    lax.fori_loop(0, ZC, zrow, 0)

    def zchunk(q, carry):
        pltpu.sync_copy(zbuf, acc.at[pl.ds(s * RPT + q * ZC, ZC)])
        return carry

    lax.fori_loop(0, RPT // ZC, zchunk, 0)
    plsc.subcore_barrier()

    def body(i, carry):
        g = 2 * i
        wait(g, *A)
        scatter(idx_a, rows_a)
        fire(g + 2, *A)
        wait(g + 1, *B)
        scatter(idx_b, rows_b)

        @pl.when(g + 3 < NB)
        def _():
            fire(g + 3, *B)

        return carry

    lax.fori_loop(0, (NB - 1) // 2, body, 0)

    wait(NB - 1, *A)
    scatter(idx_a, rows_a)

    @pl.when(s < XTRA)
    def _tail():
        row = NS * NB * SUB + s
        eoff = row * CH
        pltpu.sync_copy(seg_hbm.at[pl.ds(row, 1)], idx_a.at[pl.ds(0, 1)])
        pltpu.sync_copy(inter_hbm.at[pl.ds(eoff, CH), pl.ds(col0, DH)],
                        rows_a.at[pl.ds(0, CH)])
        pltpu.sync_copy(rows_a.at[pl.ds(0, CH)], acc.at[idx_a.at[0]], add=True)

    plsc.subcore_barrier()

    r0 = s * RPT
    pltpu.sync_copy(acc.at[pl.ds(r0, BC)], rows_a)
    w1 = pltpu.async_copy(rows_a, out_hbm.at[pl.ds(r0, BC), pl.ds(col0, DH)],
                          semr_a)
    pltpu.sync_copy(acc.at[pl.ds(r0 + BC, RB2)], zbuf.at[pl.ds(0, RB2)])
    w2 = pltpu.async_copy(zbuf.at[pl.ds(0, RB2)],
                          out_hbm.at[pl.ds(r0 + BC, RB2), pl.ds(col0, DH)],
                          semr_b)
    w1.wait()
    w2.wait()


def kernel(ind_2, inter):
    seg = ind_2[:, 0].reshape(NROWS, CH)
    return _seg_sum(seg, inter)
